# baseline (speedup 1.0000x reference)
import functools

import jax
import jax.numpy as jnp
from jax import lax
from jax.experimental import pallas as pl
from jax.experimental.pallas import tpu as pltpu
from jax.experimental.pallas import tpu_sc as plsc

NC = 2
NS = 16
LANES = 16
K = 80
NB = 5
BLK = 50


def _fast_rsqrt(x):
    i = plsc.bitcast(x, jnp.int32)
    i = 0x5F3759DF - (i >> 1)
    y = plsc.bitcast(i, jnp.float32)
    h = x * 0.5
    for _ in range(3):
        y = y * (1.5 - h * y * y)
    return jnp.where(x > 0, y, 0.0)


def _sc_gcn(x, row4, col4, ew4, zeros_a, zeros_d):
    n, d = x.shape
    dh = d // NC
    npa = zeros_a.shape[0]
    npd = zeros_d.shape[0]
    nblk = row4.shape[1]
    apt = npa // NS
    dpt = npd // NS
    mesh = plsc.VectorSubcoreMesh(core_axis_name="c", subcore_axis_name="s")

    @functools.partial(
        pl.kernel,
        out_type=(
            jax.ShapeDtypeStruct((NC, npa, dh), jnp.float32),
            jax.ShapeDtypeStruct((NC, n), jnp.float32),
            jax.ShapeDtypeStruct((NC, n, dh), jnp.bfloat16),
        ),
        mesh=mesh,
        scratch_types=[
            pltpu.VMEM((BLK, K), jnp.int32),
            pltpu.VMEM((BLK, K), jnp.int32),
            pltpu.VMEM((BLK, K), jnp.float32),
            pltpu.VMEM((BLK, K), jnp.int32),
            pltpu.VMEM((BLK, K), jnp.int32),
            pltpu.VMEM((BLK, K), jnp.float32),
            pltpu.VMEM((dpt,), jnp.float32),
            pltpu.VMEM((K,), jnp.float32),
            pltpu.VMEM((K,), jnp.float32),
        ] + [pltpu.VMEM((K, dh), jnp.bfloat16)] * NB
          + [pltpu.VMEM((K, dh), jnp.float32)] * NB + [
            pltpu.VMEM_SHARED((npa, dh), jnp.float32),
            pltpu.VMEM_SHARED((npd,), jnp.float32),
            pltpu.SemaphoreType.DMA,
            pltpu.SemaphoreType.DMA,
            pltpu.SemaphoreType.DMA,
            pltpu.SemaphoreType.DMA,
        ],
        compiler_params=pltpu.CompilerParams(use_tc_tiling_on_sc=False,
                                             needs_layout_passes=False),
    )
    def k(x_hbm, row_hbm, col_hbm, ew_hbm, za_hbm, zd_hbm,
          z_out, dis_out, y_out,
          rb0, cb0, eb0, rb1, cb1, eb1, degb, dv0, dv1, *rest):
        c = lax.axis_index("c")
        s = lax.axis_index("s")
        rsets = (rb0, rb1)
        csets = (cb0, cb1)
        esets = (eb0, eb1)
        gbufs = rest[:NB]
        sbufs = rest[NB:2 * NB]
        acc, dacc, isem, dsem, gsem, ssem = rest[2 * NB:]
        g0 = sbufs[0]
        gbf = gbufs[0]

        def issue_idx(bi, t):
            pltpu.async_copy(row_hbm.at[s, bi], rsets[t], isem)
            pltpu.async_copy(col_hbm.at[s, bi], csets[t], isem)
            pltpu.async_copy(ew_hbm.at[s, bi], esets[t], isem)

        def wait_idx(bi, t):
            pltpu.make_async_copy(row_hbm.at[s, bi], rsets[t], isem).wait()
            pltpu.make_async_copy(col_hbm.at[s, bi], csets[t], isem).wait()
            pltpu.make_async_copy(ew_hbm.at[s, bi], esets[t], isem).wait()

        pltpu.sync_copy(za_hbm.at[pl.ds(s * apt, apt)],
                        acc.at[pl.ds(s * apt, apt)])
        pltpu.sync_copy(zd_hbm.at[pl.ds(s * dpt, dpt)],
                        dacc.at[pl.ds(s * dpt, dpt)])
        issue_idx(0, 0)
        plsc.subcore_barrier()

        def deg_block(bi, carry):
            t = lax.rem(bi, 2)

            def tcase(tt):
                wait_idx(bi, tt)

                @pl.when(bi < nblk - 1)
                def _():
                    issue_idx(bi + 1, 1 - tt)

                def fire(li, cc):
                    pltpu.async_copy(esets[tt].at[li],
                                     dacc.at[csets[tt].at[li]], dsem,
                                     add=True)
                    return cc

                lax.fori_loop(0, BLK, fire, 0)

                def drain(li, cc):
                    pltpu.make_async_copy(esets[tt].at[li],
                                          dacc.at[csets[tt].at[li]],
                                          dsem).wait()
                    return cc

                lax.fori_loop(0, BLK, drain, 0)

            @pl.when(t == 0)
            def _():
                tcase(0)

            @pl.when(t == 1)
            def _():
                tcase(1)

            return carry

        with jax.named_scope("deg_phase"):
            lax.fori_loop(0, nblk, deg_block, 0)
            plsc.subcore_barrier()

        pltpu.sync_copy(dacc.at[pl.ds(s * dpt, dpt)], degb)

        def dis_group(g, carry):
            sl = pl.ds(g * LANES, LANES)
            degb[sl] = _fast_rsqrt(degb[sl])
            return carry

        with jax.named_scope("dis_phase"):
            lax.fori_loop(0, dpt // LANES, dis_group, 0)
            pltpu.sync_copy(degb, dacc.at[pl.ds(s * dpt, dpt)])
            plsc.subcore_barrier()

        @pl.when(s == 0)
        def _():
            pltpu.sync_copy(dacc.at[pl.ds(0, n)], dis_out.at[c])

        nscale = jnp.where(s == NS - 1, (n - 640 * (NS - 1)) // K, 640 // K)

        def yrows(j, carry):
            off = s * 640 + j * K
            pltpu.sync_copy(x_hbm.at[pl.ds(off, K), pl.ds(c * dh, dh)], g0)
            pltpu.sync_copy(dacc.at[pl.ds(off, K)], dv0)

            def ygroup(g, cc):
                base = g * LANES
                sv = dv0[pl.ds(base, LANES)]
                for l in range(LANES):
                    sc = sv[l]
                    r = base + l
                    for jj in range(dh // LANES):
                        sl = (r, pl.ds(jj * LANES, LANES))
                        g0[sl] = g0[sl] * sc
                return cc

            lax.fori_loop(0, K // LANES, ygroup, 0)

            def yrow(r, cc):
                for jj in range(dh // 32):
                    a = g0[r, pl.ds(jj * 32, LANES)]
                    b = g0[r, pl.ds(jj * 32 + LANES, LANES)]
                    gbf[r, pl.ds(jj * 32, 32)] = plsc.pack(a, b, format=plsc.PackFormat.INTERLEAVED)
                return cc

            lax.fori_loop(0, K, yrow, 0)
            pltpu.sync_copy(gbf, y_out.at[c, pl.ds(off, K)])
            return carry

        lax.fori_loop(0, nscale, yrows, 0)
        plsc.subcore_barrier()

        def issue_gather(rbuf, li, gb):
            pltpu.async_copy(y_out.at[c].at[rbuf.at[li]], gb, gsem)

        def wait_gather(rbuf, li, gb):
            pltpu.make_async_copy(y_out.at[c].at[rbuf.at[li]], gb,
                                  gsem).wait()

        def issue_scatter(cbuf, li, sb):
            pltpu.async_copy(sb, acc.at[cbuf.at[li]], ssem, add=True)

        def wait_scatter(cbuf, li, sb):
            pltpu.make_async_copy(sb, acc.at[cbuf.at[li]], ssem).wait()

        def scale(ebuf, li, gb, sb):
            def sgroup(g, cc):
                base = g * LANES
                sv = ebuf[li, pl.ds(base, LANES)]
                for l in range(LANES):
                    sc = sv[l]
                    r = base + l
                    for j in range(dh // 32):
                        ab = gb[r, pl.ds(j * 32, 32)]
                        a, b = plsc.unpack(ab, format=plsc.PackFormat.INTERLEAVED)
                        sb[r, pl.ds(j * 32, LANES)] = a * sc
                        sb[r, pl.ds(j * 32 + LANES, LANES)] = b * sc
                return cc

            lax.fori_loop(0, K // LANES, sgroup, 0)

        issue_idx(0, 0)

        def msg_block(bi, carry):
            t = lax.rem(bi, 2)

            def tcase(tt):
                rbuf, cbuf, ebuf = rsets[tt], csets[tt], esets[tt]
                wait_idx(bi, tt)

                @pl.when(bi < nblk - 1)
                def _():
                    issue_idx(bi + 1, 1 - tt)

                for b in range(NB):
                    issue_gather(rbuf, b, gbufs[b])

                def ring(q, cc):
                    for b in range(NB):
                        li = q * NB + b
                        gb, sb = gbufs[b], sbufs[b]
                        wait_gather(rbuf, li, gb)

                        @pl.when(q > 0)
                        def _():
                            wait_scatter(cbuf, li, sb)

                        scale(ebuf, li, gb, sb)
                        issue_gather(rbuf, li + NB, gb)
                        issue_scatter(cbuf, li, sb)
                    return cc

                lax.fori_loop(0, BLK // NB - 1, ring, 0)

                last = BLK - NB
                for b in range(NB):
                    li = last + b
                    gb, sb = gbufs[b], sbufs[b]
                    wait_gather(rbuf, li, gb)
                    wait_scatter(cbuf, li, sb)
                    scale(ebuf, li, gb, sb)
                    issue_scatter(cbuf, li, sb)
                for b in range(NB):
                    wait_scatter(cbuf, last + b, sbufs[b])

            @pl.when(t == 0)
            def _():
                tcase(0)

            @pl.when(t == 1)
            def _():
                tcase(1)

            return carry

        with jax.named_scope("msg_phase"):
            lax.fori_loop(0, nblk, msg_block, 0)
            plsc.subcore_barrier()
        pltpu.sync_copy(acc.at[pl.ds(s * apt, apt)],
                        z_out.at[c, pl.ds(s * apt, apt)])

    return k(x, row4, col4, ew4, zeros_a, zeros_d)


def _tc_out(x, W_lin, W_gcn, dis2, z):
    n, d = x.shape
    br = 2000

    def body(x_ref, wl_ref, wg_ref, dis_ref, z_ref, out_ref):
        dn = (((1,), (1,)), ((), ()))
        xb = x_ref[...]
        zp = z_ref[...]
        zb = jnp.concatenate([zp[0], zp[1]], axis=-1)
        hl = lax.dot_general(xb, wl_ref[...], dn,
                             preferred_element_type=jnp.float32)
        hz = lax.dot_general(zb, wg_ref[...], dn,
                             preferred_element_type=jnp.float32)
        out_ref[...] = hl + dis_ref[...] * hz

    return pl.pallas_call(
        body,
        grid=(n // br,),
        in_specs=[
            pl.BlockSpec((br, d), lambda i: (i, 0)),
            pl.BlockSpec((d, d), lambda i: (0, 0)),
            pl.BlockSpec((d, d), lambda i: (0, 0)),
            pl.BlockSpec((br, 1), lambda i: (i, 0)),
            pl.BlockSpec((2, br, d // NC), lambda i: (0, i, 0)),
        ],
        out_specs=pl.BlockSpec((br, d), lambda i: (i, 0)),
        out_shape=jax.ShapeDtypeStruct((n, d), jnp.float32),
    )(x, W_lin, W_gcn, dis2, z)


def kernel(x, adj_t, edge_weight, W_lin, W_gcn):
    n, d = x.shape
    e = edge_weight.shape[0]
    dh = d // NC
    nchunks = e // (NS * K)
    nblk = nchunks // BLK

    row4 = adj_t[0].astype(jnp.int32).reshape(NS, nblk, BLK, K)
    col4 = adj_t[1].astype(jnp.int32).reshape(NS, nblk, BLK, K)
    ew4 = edge_weight.astype(jnp.float32).reshape(NS, nblk, BLK, K)
    npa = ((n + 8 * NS - 1) // (8 * NS)) * (8 * NS)
    npd = ((n + 16 * NS - 1) // (16 * NS)) * (16 * NS)
    zeros_a = jnp.zeros((npa, dh), jnp.float32)
    zeros_d = jnp.zeros((npd,), jnp.float32)

    z, dis, _ = _sc_gcn(x, row4, col4, ew4, zeros_a, zeros_d)
    return _tc_out(x, W_lin, W_gcn, dis[0].reshape(n, 1), z)


# speedup vs baseline: 1.7089x; 1.7089x over previous
import functools

import jax
import jax.numpy as jnp
from jax import lax
from jax.experimental import pallas as pl
from jax.experimental.pallas import tpu as pltpu
from jax.experimental.pallas import tpu_sc as plsc

NC = 2
NS = 16
LANES = 16
K = 80
NB = 5
BLK = 50


def _fast_rsqrt(x):
    i = plsc.bitcast(x, jnp.int32)
    i = 0x5F3759DF - (i >> 1)
    y = plsc.bitcast(i, jnp.float32)
    h = x * 0.5
    for _ in range(3):
        y = y * (1.5 - h * y * y)
    return jnp.where(x > 0, y, 0.0)


def _sc_gcn(x, row4, col4, ew4, zeros_a, zeros_d):
    n, d = x.shape
    dh = d // NC
    npa = zeros_a.shape[0]
    npd = zeros_d.shape[0]
    nblk = row4.shape[1]
    apt = npa // NS
    dpt = npd // NS
    mesh = plsc.VectorSubcoreMesh(core_axis_name="c", subcore_axis_name="s")

    @functools.partial(
        pl.kernel,
        out_type=(
            jax.ShapeDtypeStruct((NC, npa, dh), jnp.float32),
            jax.ShapeDtypeStruct((NC, n), jnp.float32),
            jax.ShapeDtypeStruct((NC, n, dh), jnp.float32),
        ),
        mesh=mesh,
        scratch_types=[
            pltpu.VMEM((BLK, K), jnp.int32),
            pltpu.VMEM((BLK, K), jnp.int32),
            pltpu.VMEM((BLK, K), jnp.float32),
            pltpu.VMEM((BLK, K), jnp.int32),
            pltpu.VMEM((BLK, K), jnp.int32),
            pltpu.VMEM((BLK, K), jnp.float32),
            pltpu.VMEM((dpt,), jnp.float32),
            pltpu.VMEM((K,), jnp.float32),
            pltpu.VMEM((K,), jnp.float32),
        ] + [pltpu.VMEM((K, dh), jnp.float32)] * (2 * NB) + [
            pltpu.VMEM_SHARED((npa, dh), jnp.float32),
            pltpu.VMEM_SHARED((npd,), jnp.float32),
            pltpu.SemaphoreType.DMA,
            pltpu.SemaphoreType.DMA,
            pltpu.SemaphoreType.DMA,
            pltpu.SemaphoreType.DMA,
        ],
        compiler_params=pltpu.CompilerParams(use_tc_tiling_on_sc=False,
                                             needs_layout_passes=False),
    )
    def k(x_hbm, row_hbm, col_hbm, ew_hbm, za_hbm, zd_hbm,
          z_out, dis_out, y_out,
          rb0, cb0, eb0, rb1, cb1, eb1, degb, dv0, dv1, *rest):
        c = lax.axis_index("c")
        s = lax.axis_index("s")
        rsets = (rb0, rb1)
        csets = (cb0, cb1)
        esets = (eb0, eb1)
        gbufs = rest[:NB]
        sbufs = rest[NB:2 * NB]
        acc, dacc, isem, dsem, gsem, ssem = rest[2 * NB:]
        g0 = gbufs[0]

        def issue_idx(bi, t):
            pltpu.async_copy(row_hbm.at[s, bi], rsets[t], isem)
            pltpu.async_copy(col_hbm.at[s, bi], csets[t], isem)
            pltpu.async_copy(ew_hbm.at[s, bi], esets[t], isem)

        def wait_idx(bi, t):
            pltpu.make_async_copy(row_hbm.at[s, bi], rsets[t], isem).wait()
            pltpu.make_async_copy(col_hbm.at[s, bi], csets[t], isem).wait()
            pltpu.make_async_copy(ew_hbm.at[s, bi], esets[t], isem).wait()

        pltpu.sync_copy(za_hbm.at[pl.ds(s * apt, apt)],
                        acc.at[pl.ds(s * apt, apt)])
        pltpu.sync_copy(zd_hbm.at[pl.ds(s * dpt, dpt)],
                        dacc.at[pl.ds(s * dpt, dpt)])
        issue_idx(0, 0)
        plsc.subcore_barrier()

        def deg_block(bi, carry):
            t = lax.rem(bi, 2)

            def tcase(tt):
                wait_idx(bi, tt)

                @pl.when(bi < nblk - 1)
                def _():
                    issue_idx(bi + 1, 1 - tt)

                def fire(li, cc):
                    pltpu.async_copy(esets[tt].at[li],
                                     dacc.at[csets[tt].at[li]], dsem,
                                     add=True)
                    return cc

                lax.fori_loop(0, BLK, fire, 0)

                def drain(li, cc):
                    pltpu.make_async_copy(esets[tt].at[li],
                                          dacc.at[csets[tt].at[li]],
                                          dsem).wait()
                    return cc

                lax.fori_loop(0, BLK, drain, 0)

            @pl.when(t == 0)
            def _():
                tcase(0)

            @pl.when(t == 1)
            def _():
                tcase(1)

            return carry

        with jax.named_scope("deg_phase"):
            lax.fori_loop(0, nblk, deg_block, 0)
            plsc.subcore_barrier()

        pltpu.sync_copy(dacc.at[pl.ds(s * dpt, dpt)], degb)

        def dis_group(g, carry):
            sl = pl.ds(g * LANES, LANES)
            degb[sl] = _fast_rsqrt(degb[sl])
            return carry

        with jax.named_scope("dis_phase"):
            lax.fori_loop(0, dpt // LANES, dis_group, 0)
            pltpu.sync_copy(degb, dacc.at[pl.ds(s * dpt, dpt)])
            plsc.subcore_barrier()

        @pl.when(s == 0)
        def _():
            pltpu.sync_copy(dacc.at[pl.ds(0, n)], dis_out.at[c])

        nscale = jnp.where(s == NS - 1, (n - 640 * (NS - 1)) // K, 640 // K)

        def yrows(j, carry):
            off = s * 640 + j * K
            pltpu.sync_copy(x_hbm.at[pl.ds(off, K), pl.ds(c * dh, dh)], g0)
            pltpu.sync_copy(dacc.at[pl.ds(off, K)], dv0)

            def ygroup(g, cc):
                base = g * LANES
                sv = dv0[pl.ds(base, LANES)]
                for l in range(LANES):
                    sc = sv[l]
                    r = base + l
                    for jj in range(dh // LANES):
                        sl = (r, pl.ds(jj * LANES, LANES))
                        g0[sl] = g0[sl] * sc
                return cc

            lax.fori_loop(0, K // LANES, ygroup, 0)
            pltpu.sync_copy(g0, y_out.at[c, pl.ds(off, K)])
            return carry

        lax.fori_loop(0, nscale, yrows, 0)
        plsc.subcore_barrier()

        def issue_gather(rbuf, li, gb):
            pltpu.async_copy(y_out.at[c].at[rbuf.at[li]], gb, gsem)

        def wait_gather(rbuf, li, gb):
            pltpu.make_async_copy(y_out.at[c].at[rbuf.at[li]], gb,
                                  gsem).wait()

        def issue_scatter(cbuf, li, sb):
            pltpu.async_copy(sb, acc.at[cbuf.at[li]], ssem, add=True)

        def wait_scatter(cbuf, li, sb):
            pltpu.make_async_copy(sb, acc.at[cbuf.at[li]], ssem).wait()

        def scale(ebuf, li, gb, sb):
            def sgroup(g, cc):
                base = g * LANES
                sv = ebuf[li, pl.ds(base, LANES)]
                for l in range(LANES):
                    sc = sv[l]
                    r = base + l
                    for j in range(dh // LANES):
                        sl = (r, pl.ds(j * LANES, LANES))
                        sb[sl] = gb[sl] * sc
                return cc

            lax.fori_loop(0, K // LANES, sgroup, 0)

        issue_idx(0, 0)

        def msg_block(bi, carry):
            t = lax.rem(bi, 2)

            def tcase(tt):
                rbuf, cbuf, ebuf = rsets[tt], csets[tt], esets[tt]
                wait_idx(bi, tt)

                @pl.when(bi < nblk - 1)
                def _():
                    issue_idx(bi + 1, 1 - tt)

                for b in range(NB):
                    issue_gather(rbuf, b, gbufs[b])

                def ring(q, cc):
                    for b in range(NB):
                        li = q * NB + b
                        gb, sb = gbufs[b], sbufs[b]
                        wait_gather(rbuf, li, gb)

                        @pl.when(q > 0)
                        def _():
                            wait_scatter(cbuf, li, sb)

                        scale(ebuf, li, gb, sb)
                        issue_gather(rbuf, li + NB, gb)
                        issue_scatter(cbuf, li, sb)
                    return cc

                lax.fori_loop(0, BLK // NB - 1, ring, 0)

                last = BLK - NB
                for b in range(NB):
                    li = last + b
                    gb, sb = gbufs[b], sbufs[b]
                    wait_gather(rbuf, li, gb)
                    wait_scatter(cbuf, li, sb)
                    scale(ebuf, li, gb, sb)
                    issue_scatter(cbuf, li, sb)
                for b in range(NB):
                    wait_scatter(cbuf, last + b, sbufs[b])

            @pl.when(t == 0)
            def _():
                tcase(0)

            @pl.when(t == 1)
            def _():
                tcase(1)

            return carry

        with jax.named_scope("msg_phase"):
            lax.fori_loop(0, nblk, msg_block, 0)
            plsc.subcore_barrier()
        pltpu.sync_copy(acc.at[pl.ds(s * apt, apt)],
                        z_out.at[c, pl.ds(s * apt, apt)])

    return k(x, row4, col4, ew4, zeros_a, zeros_d)


def _tc_out(x, W_lin, W_gcn, dis2, z):
    n, d = x.shape
    br = 2000

    def body(x_ref, wl_ref, wg_ref, dis_ref, z_ref, out_ref):
        dn = (((1,), (1,)), ((), ()))
        xb = x_ref[...]
        zp = z_ref[...]
        zb = jnp.concatenate([zp[0], zp[1]], axis=-1)
        hl = lax.dot_general(xb, wl_ref[...], dn,
                             preferred_element_type=jnp.float32)
        hz = lax.dot_general(zb, wg_ref[...], dn,
                             preferred_element_type=jnp.float32)
        out_ref[...] = hl + dis_ref[...] * hz

    return pl.pallas_call(
        body,
        grid=(n // br,),
        in_specs=[
            pl.BlockSpec((br, d), lambda i: (i, 0)),
            pl.BlockSpec((d, d), lambda i: (0, 0)),
            pl.BlockSpec((d, d), lambda i: (0, 0)),
            pl.BlockSpec((br, 1), lambda i: (i, 0)),
            pl.BlockSpec((2, br, d // NC), lambda i: (0, i, 0)),
        ],
        out_specs=pl.BlockSpec((br, d), lambda i: (i, 0)),
        out_shape=jax.ShapeDtypeStruct((n, d), jnp.float32),
    )(x, W_lin, W_gcn, dis2, z)


def kernel(x, adj_t, edge_weight, W_lin, W_gcn):
    n, d = x.shape
    e = edge_weight.shape[0]
    dh = d // NC
    nchunks = e // (NS * K)
    nblk = nchunks // BLK

    row4 = adj_t[0].astype(jnp.int32).reshape(NS, nblk, BLK, K)
    col4 = adj_t[1].astype(jnp.int32).reshape(NS, nblk, BLK, K)
    ew4 = edge_weight.astype(jnp.float32).reshape(NS, nblk, BLK, K)
    npa = ((n + 8 * NS - 1) // (8 * NS)) * (8 * NS)
    npd = ((n + 16 * NS - 1) // (16 * NS)) * (16 * NS)
    zeros_a = jnp.zeros((npa, dh), jnp.float32)
    zeros_d = jnp.zeros((npd,), jnp.float32)

    z, dis, _ = _sc_gcn(x, row4, col4, ew4, zeros_a, zeros_d)
    return _tc_out(x, W_lin, W_gcn, dis[0].reshape(n, 1), z)
